# baseline (speedup 1.0000x reference)
import jax
import jax.numpy as jnp
from jax import lax
from jax.experimental import pallas as pl
from jax.experimental.pallas import tpu as pltpu
from jax.experimental.pallas import tpu_sc as plsc

VOCAB = 1000
N_TOK = 51200
NC, NS = 2, 16
NW = NC * NS
LSE_PAD = 1024

ROWS_PER_W = N_TOK // NW
CHUNK = 32
N_CHUNKS = ROWS_PER_W // CHUNK

LW = N_TOK // NW
LG = LW // 16
LD = 80
N_LD = LW // LD

N_VT = VOCAB // 8
N_PT = N_TOK // 128


def _lse_body(x_ref, lse_ref, tpad_ref):
    x = x_ref[...]
    m = jnp.max(x, axis=1)
    s = jnp.sum(jnp.exp(x - m[:, None]), axis=1)
    lse = m + jnp.log(s)
    lse_ref[...] = jnp.concatenate(
        [lse, jnp.zeros((LSE_PAD - VOCAB,), jnp.float32)]
    )[:, None]
    tpad_ref[...] = jnp.concatenate(
        [x, jnp.zeros((VOCAB, 1024 - VOCAB), jnp.float32)], axis=1
    )


@jax.jit
def _lse_call(table):
    return pl.pallas_call(
        _lse_body,
        out_shape=(
            jax.ShapeDtypeStruct((LSE_PAD, 1), jnp.float32),
            jax.ShapeDtypeStruct((VOCAB, 1024), jnp.float32),
        ),
    )(table)


def _sc_body(table, table1m, idxr, tf, lse, out, partials,
             idx_v, buf, lse_v, tl_v, lin_v, vals_v, acc,
             semg, sems, semv):
    c_id = lax.axis_index("c")
    s_id = lax.axis_index("s")
    wid = s_id * NC + c_id
    base = wid * ROWS_PER_W
    pltpu.sync_copy(idxr.at[wid], idx_v)

    def gather_desc(c, b):
        return pltpu.make_async_copy(
            table.at[idx_v.at[c]], buf.at[b], semg.at[b]
        )

    def scatter_desc(c, b):
        return pltpu.make_async_copy(
            buf.at[b], out.at[pl.ds(base + c * CHUNK, CHUNK)], sems.at[b]
        )

    gather_desc(0, 0).start()

    def step(k, carry):
        for b in range(2):
            c = 2 * k + b
            ob = 1 - b
            gather_desc(c, b).wait()

            @pl.when(c + 1 < N_CHUNKS)
            def _start_next():
                @pl.when(c >= 1)
                def _drain():
                    scatter_desc(c - 1, ob).wait()

                gather_desc(c + 1, ob).start()

            scatter_desc(c, b).start()
        return carry

    lax.fori_loop(0, N_CHUNKS // 2, step, 0)
    scatter_desc(N_CHUNKS - 2, 0).wait()
    scatter_desc(N_CHUNKS - 1, 1).wait()

    pltpu.sync_copy(lse, lse_v)
    pltpu.sync_copy(tf.at[pl.ds(base, LW)], tl_v)

    @plsc.parallel_loop(0, LG, unroll=4)
    def build_lin(m):
        iv = idx_v[m // 2, pl.ds((m % 2) * 16, 16)]
        tv = tl_v[pl.ds(m * 16, 16)]
        lin_v[m // 5, pl.ds((m % 5) * 16, 16)] = iv * VOCAB + tv

    for d in range(N_LD):
        pltpu.async_copy(table1m.at[lin_v.at[d]], vals_v.at[d], semv)
    for d in range(N_LD):
        pltpu.make_async_copy(
            table1m.at[lin_v.at[d]], vals_v.at[d], semv
        ).wait()

    acc[...] = jnp.zeros((16,), jnp.float32)
    zeros16 = jnp.zeros((16,), jnp.int32)
    ios = lax.iota(jnp.int32, 16)

    def accum(m, carry):
        iv = idx_v[m // 2, pl.ds((m % 2) * 16, 16)]
        d = m // 5
        o = (m % 5) * 16
        vals = plsc.load_gather(
            vals_v, [jnp.full((16,), 1, jnp.int32) * d, o + ios, zeros16]
        )
        lsev = plsc.load_gather(lse_v, [iv])
        acc[...] = acc[...] + (lsev - vals)
        return carry

    lax.fori_loop(0, LG, accum, 0)
    pltpu.sync_copy(acc, partials.at[wid])


@jax.jit
def _sc_call(table, table1m, idx_r, t_f, lse_flat):
    mesh = plsc.VectorSubcoreMesh(
        core_axis_name="c", subcore_axis_name="s", num_cores=NC,
        num_subcores=NS,
    )
    return pl.kernel(
        _sc_body,
        out_type=(
            jax.ShapeDtypeStruct((N_TOK, 1024), jnp.float32),
            jax.ShapeDtypeStruct((NW, 16), jnp.float32),
        ),
        mesh=mesh,
        compiler_params=pltpu.CompilerParams(
            use_tc_tiling_on_sc=False, needs_layout_passes=False
        ),
        scratch_types=[
            pltpu.VMEM((N_CHUNKS, CHUNK), jnp.int32),
            pltpu.VMEM((2, CHUNK, 1024), jnp.float32),
            pltpu.VMEM((LSE_PAD,), jnp.float32),
            pltpu.VMEM((LW,), jnp.int32),
            pltpu.VMEM((N_LD, LD), jnp.int32),
            pltpu.VMEM((N_LD, LD, 1), jnp.float32),
            pltpu.VMEM((16,), jnp.float32),
            pltpu.SemaphoreType.DMA((2,)),
            pltpu.SemaphoreType.DMA((2,)),
            pltpu.SemaphoreType.DMA,
        ],
    )(table, table1m, idx_r, t_f, lse_flat)


def _fmt_body(x_ref, o_ref):
    x = x_ref[...]
    z = x.reshape(128, 1024).T
    o_ref[...] = z[:VOCAB].reshape(N_VT, 1, 8, 128)


@jax.jit
def _fmt_call(x3):
    return pl.pallas_call(
        _fmt_body,
        out_shape=jax.ShapeDtypeStruct((N_VT, N_PT, 8, 128), jnp.float32),
        grid=(N_PT,),
        in_specs=[pl.BlockSpec((1024, 128), lambda i: (i, 0))],
        out_specs=pl.BlockSpec((N_VT, 1, 8, 128), lambda i: (0, i, 0, 0)),
    )(x3)


def _loss_body(p_ref, o_ref):
    o_ref[...] = (jnp.sum(p_ref[...]) / N_TOK).reshape(1, 1)


@jax.jit
def _loss_call(partials):
    return pl.pallas_call(
        _loss_body,
        out_shape=jax.ShapeDtypeStruct((1, 1), jnp.float32),
    )(partials)


def kernel(idx, targets, token_emb):
    idx_r = idx.reshape(NW, N_CHUNKS, CHUNK).astype(jnp.int32)
    t_f = targets.reshape(-1).astype(jnp.int32)
    lse, tpad = _lse_call(token_emb)
    lin, partials = _sc_call(
        tpad, token_emb.reshape(VOCAB * VOCAB, 1), idx_r, t_f,
        lse.reshape(LSE_PAD),
    )
    out4 = _fmt_call(lin.reshape(N_TOK * 8, 128))
    logits2 = out4.transpose(1, 3, 0, 2).reshape(N_TOK, VOCAB)
    loss = _loss_call(partials)[0, 0]
    return logits2, loss


# speedup vs baseline: 3.6111x; 3.6111x over previous
import jax
import jax.numpy as jnp
from jax import lax
from jax.experimental import pallas as pl
from jax.experimental.pallas import tpu as pltpu
from jax.experimental.pallas import tpu_sc as plsc

VOCAB = 1000
N_TOK = 51200
NC, NS = 2, 16
NW = NC * NS
LSE_PAD = 1024

ROWS_PER_W = N_TOK // NW
CHUNK = 32
N_CHUNKS = ROWS_PER_W // CHUNK

LW = N_TOK // NW
LG = LW // 16
LD = 80
N_LD = LW // LD

N_VT = VOCAB // 8
N_PT = N_TOK // 128


def _lse_body(x_ref, lse_ref, tpad_ref):
    x = x_ref[...]
    m = jnp.max(x, axis=1)
    s = jnp.sum(jnp.exp(x - m[:, None]), axis=1)
    lse = m + jnp.log(s)
    lse_ref[...] = jnp.concatenate(
        [lse, jnp.zeros((LSE_PAD - VOCAB,), jnp.float32)]
    )[:, None]
    tpad_ref[...] = jnp.concatenate(
        [x, jnp.zeros((VOCAB, 1024 - VOCAB), jnp.float32)], axis=1
    )


@jax.jit
def _lse_call(table):
    return pl.pallas_call(
        _lse_body,
        out_shape=(
            jax.ShapeDtypeStruct((LSE_PAD, 1), jnp.float32),
            jax.ShapeDtypeStruct((VOCAB, 1024), jnp.float32),
        ),
    )(table)


def _sc_body(table, table1m, idxr, tf, lse, out, partials,
             idx_v, buf, lse_v, tl_v, lin_v, vals_v, acc,
             semg, sems, semv):
    c_id = lax.axis_index("c")
    s_id = lax.axis_index("s")
    wid = s_id * NC + c_id
    base = wid * ROWS_PER_W
    pltpu.sync_copy(idxr.at[wid], idx_v)

    def gather_desc(c, b):
        return pltpu.make_async_copy(
            table.at[idx_v.at[c]], buf.at[b], semg.at[b]
        )

    def scatter_desc(c, b):
        return pltpu.make_async_copy(
            buf.at[b], out.at[pl.ds(base + c * CHUNK, CHUNK)], sems.at[b]
        )

    gather_desc(0, 0).start()

    def step(k, carry):
        for b in range(2):
            c = 2 * k + b
            ob = 1 - b
            gather_desc(c, b).wait()

            @pl.when(c + 1 < N_CHUNKS)
            def _start_next():
                @pl.when(c >= 1)
                def _drain():
                    scatter_desc(c - 1, ob).wait()

                gather_desc(c + 1, ob).start()

            scatter_desc(c, b).start()
        return carry

    lax.fori_loop(0, N_CHUNKS // 2, step, 0)
    scatter_desc(N_CHUNKS - 2, 0).wait()
    scatter_desc(N_CHUNKS - 1, 1).wait()

    pltpu.sync_copy(lse, lse_v)
    pltpu.sync_copy(tf.at[pl.ds(base, LW)], tl_v)

    def build_lin(m, carry):
        iv = idx_v[m // 2, pl.ds((m % 2) * 16, 16)]
        tv = tl_v[pl.ds(m * 16, 16)]
        lin = iv * VOCAB + tv
        lin_v[m // 5, pl.ds((m % 5) * 16, 16)] = lax.shift_right_logical(
            lin, 4
        )
        return carry

    lax.fori_loop(0, LG, build_lin, 0)

    for d in range(N_LD):
        pltpu.async_copy(table1m.at[lin_v.at[d]], vals_v.at[d], semv)
    for d in range(N_LD):
        pltpu.make_async_copy(
            table1m.at[lin_v.at[d]], vals_v.at[d], semv
        ).wait()

    acc[...] = jnp.zeros((16,), jnp.float32)
    ios = lax.iota(jnp.int32, 16)

    def accum(m, carry):
        iv = idx_v[m // 2, pl.ds((m % 2) * 16, 16)]
        tv = tl_v[pl.ds(m * 16, 16)]
        fmod = jnp.bitwise_and(iv * VOCAB + tv, 15)
        d = m // 5
        o = (m % 5) * 16
        vals = plsc.load_gather(
            vals_v, [jnp.full((16,), 1, jnp.int32) * d, o + ios, fmod]
        )
        lsev = plsc.load_gather(lse_v, [iv])
        acc[...] = acc[...] + (lsev - vals)
        return carry

    lax.fori_loop(0, LG, accum, 0)
    pltpu.sync_copy(acc, partials.at[wid])


@jax.jit
def _sc_call(table, table1m, idx_r, t_f, lse_flat):
    mesh = plsc.VectorSubcoreMesh(
        core_axis_name="c", subcore_axis_name="s", num_cores=NC,
        num_subcores=NS,
    )
    return pl.kernel(
        _sc_body,
        out_type=(
            jax.ShapeDtypeStruct((N_TOK, 1024), jnp.float32),
            jax.ShapeDtypeStruct((NW, 16), jnp.float32),
        ),
        mesh=mesh,
        compiler_params=pltpu.CompilerParams(
            use_tc_tiling_on_sc=False, needs_layout_passes=False
        ),
        scratch_types=[
            pltpu.VMEM((N_CHUNKS, CHUNK), jnp.int32),
            pltpu.VMEM((2, CHUNK, 1024), jnp.float32),
            pltpu.VMEM((LSE_PAD,), jnp.float32),
            pltpu.VMEM((LW,), jnp.int32),
            pltpu.VMEM((N_LD, LD), jnp.int32),
            pltpu.VMEM((N_LD, LD, 16), jnp.float32),
            pltpu.VMEM((16,), jnp.float32),
            pltpu.SemaphoreType.DMA((2,)),
            pltpu.SemaphoreType.DMA((2,)),
            pltpu.SemaphoreType.DMA,
        ],
    )(table, table1m, idx_r, t_f, lse_flat)


FPT = 4


def _fmt_body(x_ref, o_ref):
    x = x_ref[...]
    z = x.reshape(FPT * 128, 1024).T
    o_ref[...] = z[:VOCAB].reshape(N_VT, 8, FPT, 128).transpose(0, 2, 1, 3)


@jax.jit
def _fmt_call(x3):
    return pl.pallas_call(
        _fmt_body,
        out_shape=jax.ShapeDtypeStruct((N_VT, N_PT, 8, 128), jnp.float32),
        grid=(N_PT // FPT,),
        in_specs=[pl.BlockSpec((FPT * 1024, 128), lambda i: (i, 0))],
        out_specs=pl.BlockSpec(
            (N_VT, FPT, 8, 128), lambda i: (0, i, 0, 0)
        ),
    )(x3)


def _loss_body(p_ref, o_ref):
    o_ref[...] = (jnp.sum(p_ref[...]) / N_TOK).reshape(1, 1)


@jax.jit
def _loss_call(partials):
    return pl.pallas_call(
        _loss_body,
        out_shape=jax.ShapeDtypeStruct((1, 1), jnp.float32),
    )(partials)


def kernel(idx, targets, token_emb):
    idx_r = idx.reshape(NW, N_CHUNKS, CHUNK).astype(jnp.int32)
    t_f = targets.reshape(-1).astype(jnp.int32)
    lse, tpad = _lse_call(token_emb)
    lin, partials = _sc_call(
        tpad, token_emb.reshape(VOCAB * VOCAB // 16, 16), idx_r, t_f,
        lse.reshape(LSE_PAD),
    )
    out4 = _fmt_call(lin.reshape(N_TOK * 8, 128))
    logits2 = out4.transpose(1, 3, 0, 2).reshape(N_TOK, VOCAB)
    loss = _loss_call(partials)[0, 0]
    return logits2, loss


# speedup vs baseline: 3.8579x; 1.0683x over previous
import jax
import jax.numpy as jnp
from jax import lax
from jax.experimental import pallas as pl
from jax.experimental.pallas import tpu as pltpu
from jax.experimental.pallas import tpu_sc as plsc

VOCAB = 1000
N_TOK = 51200
NC, NS = 2, 16
NW = NC * NS
LSE_PAD = 1024

ROWS_PER_W = N_TOK // NW
CHUNK = 32
N_CHUNKS = ROWS_PER_W // CHUNK

LW = N_TOK // NW
LG = LW // 16
LD = 80
N_LD = LW // LD

N_VT = VOCAB // 8
N_PT = N_TOK // 128


def _lse_body(x_ref, lse_ref, tpad_ref):
    x = x_ref[...]
    m = jnp.max(x, axis=1)
    s = jnp.sum(jnp.exp(x - m[:, None]), axis=1)
    lse = m + jnp.log(s)
    lse_ref[...] = jnp.concatenate(
        [lse, jnp.zeros((LSE_PAD - VOCAB,), jnp.float32)]
    )[:, None]
    tpad_ref[...] = jnp.concatenate(
        [x, jnp.zeros((VOCAB, 1024 - VOCAB), jnp.float32)], axis=1
    )


@jax.jit
def _lse_call(table):
    return pl.pallas_call(
        _lse_body,
        out_shape=(
            jax.ShapeDtypeStruct((LSE_PAD, 1), jnp.float32),
            jax.ShapeDtypeStruct((VOCAB, 1024), jnp.float32),
        ),
    )(table)


def _sc_body(table, table1m, idxr, tf, lse, out, partials,
             idx_v, buf, lse_v, tl_v, lin_v, vals_v, acc,
             semg, sems, semv):
    c_id = lax.axis_index("c")
    s_id = lax.axis_index("s")
    wid = s_id * NC + c_id
    base = wid * ROWS_PER_W
    pltpu.sync_copy(idxr.at[wid], idx_v)

    def gather_desc(c, b):
        return pltpu.make_async_copy(
            table.at[idx_v.at[c]], buf.at[b], semg.at[b]
        )

    def scatter_desc(c, b):
        return pltpu.make_async_copy(
            buf.at[b], out.at[pl.ds(base + c * CHUNK, CHUNK)], sems.at[b]
        )

    gather_desc(0, 0).start()

    def step(k, carry):
        for b in range(2):
            c = 2 * k + b
            ob = 1 - b
            gather_desc(c, b).wait()

            @pl.when(c + 1 < N_CHUNKS)
            def _start_next():
                @pl.when(c >= 1)
                def _drain():
                    scatter_desc(c - 1, ob).wait()

                gather_desc(c + 1, ob).start()

            scatter_desc(c, b).start()
        return carry

    lax.fori_loop(0, N_CHUNKS // 2, step, 0)
    scatter_desc(N_CHUNKS - 2, 0).wait()
    scatter_desc(N_CHUNKS - 1, 1).wait()

    pltpu.sync_copy(lse, lse_v)
    pltpu.sync_copy(tf.at[pl.ds(base, LW)], tl_v)

    def build_lin(m, carry):
        iv = idx_v[m // 2, pl.ds((m % 2) * 16, 16)]
        tv = tl_v[pl.ds(m * 16, 16)]
        lin = iv * VOCAB + tv
        lin_v[m // 5, pl.ds((m % 5) * 16, 16)] = lax.shift_right_logical(
            lin, 4
        )
        return carry

    lax.fori_loop(0, LG, build_lin, 0)

    for d in range(N_LD):
        pltpu.async_copy(table1m.at[lin_v.at[d]], vals_v.at[d], semv)
    for d in range(N_LD):
        pltpu.make_async_copy(
            table1m.at[lin_v.at[d]], vals_v.at[d], semv
        ).wait()

    acc[...] = jnp.zeros((16,), jnp.float32)
    ios = lax.iota(jnp.int32, 16)

    def accum(m, carry):
        iv = idx_v[m // 2, pl.ds((m % 2) * 16, 16)]
        tv = tl_v[pl.ds(m * 16, 16)]
        fmod = jnp.bitwise_and(iv * VOCAB + tv, 15)
        d = m // 5
        o = (m % 5) * 16
        vals = plsc.load_gather(
            vals_v, [jnp.full((16,), 1, jnp.int32) * d, o + ios, fmod]
        )
        lsev = plsc.load_gather(lse_v, [iv])
        acc[...] = acc[...] + (lsev - vals)
        return carry

    lax.fori_loop(0, LG, accum, 0)
    pltpu.sync_copy(acc, partials.at[wid])


@jax.jit
def _sc_call(table, table1m, idx_r, t_f, lse_flat):
    mesh = plsc.VectorSubcoreMesh(
        core_axis_name="c", subcore_axis_name="s", num_cores=NC,
        num_subcores=NS,
    )
    return pl.kernel(
        _sc_body,
        out_type=(
            jax.ShapeDtypeStruct((N_TOK, 1024), jnp.float32),
            jax.ShapeDtypeStruct((NW, 16), jnp.float32),
        ),
        mesh=mesh,
        compiler_params=pltpu.CompilerParams(
            use_tc_tiling_on_sc=False, needs_layout_passes=False
        ),
        scratch_types=[
            pltpu.VMEM((N_CHUNKS, CHUNK), jnp.int32),
            pltpu.VMEM((2, CHUNK, 1024), jnp.float32),
            pltpu.VMEM((LSE_PAD,), jnp.float32),
            pltpu.VMEM((LW,), jnp.int32),
            pltpu.VMEM((N_LD, LD), jnp.int32),
            pltpu.VMEM((N_LD, LD, 16), jnp.float32),
            pltpu.VMEM((16,), jnp.float32),
            pltpu.SemaphoreType.DMA((2,)),
            pltpu.SemaphoreType.DMA((2,)),
            pltpu.SemaphoreType.DMA,
        ],
    )(table, table1m, idx_r, t_f, lse_flat)


FPT = 8


def _fmt_body(x_ref, o_ref):
    x = x_ref[...]
    z = x.reshape(FPT * 128, 1024).T
    o_ref[...] = z[:VOCAB].reshape(N_VT, 8, FPT, 128).transpose(0, 2, 1, 3)


@jax.jit
def _fmt_call(x3):
    return pl.pallas_call(
        _fmt_body,
        out_shape=jax.ShapeDtypeStruct((N_VT, N_PT, 8, 128), jnp.float32),
        grid=(N_PT // FPT,),
        in_specs=[pl.BlockSpec((FPT * 1024, 128), lambda i: (i, 0))],
        out_specs=pl.BlockSpec(
            (N_VT, FPT, 8, 128), lambda i: (0, i, 0, 0)
        ),
    )(x3)


def _loss_body(p_ref, o_ref):
    o_ref[...] = (jnp.sum(p_ref[...]) / N_TOK).reshape(1, 1)


@jax.jit
def _loss_call(partials):
    return pl.pallas_call(
        _loss_body,
        out_shape=jax.ShapeDtypeStruct((1, 1), jnp.float32),
    )(partials)


def kernel(idx, targets, token_emb):
    idx_r = idx.reshape(NW, N_CHUNKS, CHUNK).astype(jnp.int32)
    t_f = targets.reshape(-1).astype(jnp.int32)
    lse, tpad = _lse_call(token_emb)
    lin, partials = _sc_call(
        tpad, token_emb.reshape(VOCAB * VOCAB // 16, 16), idx_r, t_f,
        lse.reshape(LSE_PAD),
    )
    out4 = _fmt_call(lin.reshape(N_TOK * 8, 128))
    logits2 = out4.transpose(1, 3, 0, 2).reshape(N_TOK, VOCAB)
    loss = _loss_call(partials)[0, 0]
    return logits2, loss


# speedup vs baseline: 3.8945x; 1.0095x over previous
import functools

import jax
import jax.numpy as jnp
from jax import lax
from jax.experimental import pallas as pl
from jax.experimental.pallas import tpu as pltpu
from jax.experimental.pallas import tpu_sc as plsc

VOCAB = 1000
N_TOK = 51200
NC, NS = 2, 16
NW = NC * NS
LSE_PAD = 1024

H = 2
N_POS = N_TOK // H
ROWS_PER_W = N_POS // NW
CHUNK = 40
N_CHUNKS = ROWS_PER_W // CHUNK

LW = ROWS_PER_W
LG = LW // 16
LD = 80
N_LD = LW // LD

N_VT = VOCAB // 8
N_PT = N_TOK // 128
FPT = 8
PT_H = N_PT // H


def _lse_body(x_ref, lse_ref, tpad_ref):
    x = x_ref[...]
    m = jnp.max(x, axis=1)
    s = jnp.sum(jnp.exp(x - m[:, None]), axis=1)
    lse = m + jnp.log(s)
    lse_ref[...] = jnp.concatenate(
        [lse, jnp.zeros((LSE_PAD - VOCAB,), jnp.float32)]
    )[:, None]
    tpad_ref[...] = jnp.concatenate(
        [x, jnp.zeros((VOCAB, 1024 - VOCAB), jnp.float32)], axis=1
    )


@jax.jit
def _lse_call(table):
    return pl.pallas_call(
        _lse_body,
        out_shape=(
            jax.ShapeDtypeStruct((LSE_PAD, 1), jnp.float32),
            jax.ShapeDtypeStruct((VOCAB, 1024), jnp.float32),
        ),
    )(table)


def _sc_body(table, table16, idxw, tf, lse, out, partials,
             buf, lse_v, idxl_v, tl_v, lin_v, vals_v, acc,
             semg, sems, semv):
    c_id = lax.axis_index("c")
    s_id = lax.axis_index("s")
    wid = s_id * NC + c_id
    base = wid * ROWS_PER_W
    pltpu.sync_copy(idxw.at[wid], idxl_v)

    def gather_desc(c, b):
        return pltpu.make_async_copy(
            table.at[idxl_v.at[pl.ds(c * CHUNK, CHUNK)]], buf.at[b],
            semg.at[b]
        )

    def scatter_desc(c, b):
        return pltpu.make_async_copy(
            buf.at[b], out.at[pl.ds(base + c * CHUNK, CHUNK)], sems.at[b]
        )

    gather_desc(0, 0).start()

    def step(k, carry):
        for b in range(2):
            c = 2 * k + b
            ob = 1 - b
            gather_desc(c, b).wait()

            @pl.when(c + 1 < N_CHUNKS)
            def _start_next():
                @pl.when(c >= 1)
                def _drain():
                    scatter_desc(c - 1, ob).wait()

                gather_desc(c + 1, ob).start()

            scatter_desc(c, b).start()
        return carry

    lax.fori_loop(0, N_CHUNKS // 2, step, 0)
    scatter_desc(N_CHUNKS - 2, 0).wait()
    scatter_desc(N_CHUNKS - 1, 1).wait()

    pltpu.sync_copy(lse, lse_v)
    pltpu.sync_copy(tf.at[pl.ds(base, LW)], tl_v)

    def build_lin(m, carry):
        iv = idxl_v[pl.ds(m * 16, 16)]
        tv = tl_v[pl.ds(m * 16, 16)]
        lin_v[m // 5, pl.ds((m % 5) * 16, 16)] = lax.shift_right_logical(
            iv * VOCAB + tv, 4
        )
        return carry

    lax.fori_loop(0, LG, build_lin, 0)

    for d in range(N_LD):
        pltpu.async_copy(table16.at[lin_v.at[d]], vals_v.at[d], semv)
    for d in range(N_LD):
        pltpu.make_async_copy(
            table16.at[lin_v.at[d]], vals_v.at[d], semv
        ).wait()

    acc[...] = jnp.zeros((16,), jnp.float32)
    ios = lax.iota(jnp.int32, 16)

    def accum(m, carry):
        iv = idxl_v[pl.ds(m * 16, 16)]
        tv = tl_v[pl.ds(m * 16, 16)]
        fmod = jnp.bitwise_and(iv * VOCAB + tv, 15)
        d = m // 5
        o = (m % 5) * 16
        vals = plsc.load_gather(
            vals_v, [jnp.full((16,), 1, jnp.int32) * d, o + ios, fmod]
        )
        lsev = plsc.load_gather(lse_v, [iv])
        acc[...] = acc[...] + (lsev - vals)
        return carry

    lax.fori_loop(0, LG, accum, 0)
    pltpu.sync_copy(acc, partials.at[wid])


@jax.jit
def _sc_call(table, table16, idx_w, t_f, lse_flat):
    mesh = plsc.VectorSubcoreMesh(
        core_axis_name="c", subcore_axis_name="s", num_cores=NC,
        num_subcores=NS,
    )
    return pl.kernel(
        _sc_body,
        out_type=(
            jax.ShapeDtypeStruct((N_POS, 1024), jnp.float32),
            jax.ShapeDtypeStruct((NW, 16), jnp.float32),
        ),
        mesh=mesh,
        compiler_params=pltpu.CompilerParams(
            use_tc_tiling_on_sc=False, needs_layout_passes=False
        ),
        scratch_types=[
            pltpu.VMEM((2, CHUNK, 1024), jnp.float32),
            pltpu.VMEM((LSE_PAD,), jnp.float32),
            pltpu.VMEM((LW,), jnp.int32),
            pltpu.VMEM((LW,), jnp.int32),
            pltpu.VMEM((N_LD, LD), jnp.int32),
            pltpu.VMEM((N_LD, LD, 16), jnp.float32),
            pltpu.VMEM((16,), jnp.float32),
            pltpu.SemaphoreType.DMA((2,)),
            pltpu.SemaphoreType.DMA((2,)),
            pltpu.SemaphoreType.DMA,
        ],
    )(table, table16, idx_w, t_f, lse_flat)


def _fmt_body(x_ref, o_ref):
    x = x_ref[...]
    z = x.reshape(FPT * 128, 1024).T
    o_ref[...] = z[:VOCAB].reshape(N_VT, 8, FPT, 128).transpose(0, 2, 1, 3)


def _fmt_next_body(x_ref, o_prev_ref, o_ref):
    del o_prev_ref
    _fmt_body(x_ref, o_ref)


@functools.partial(jax.jit, static_argnums=(2,), donate_argnums=(1,))
def _fmt_next_call(x3, o_prev, h):
    off = h * (PT_H // FPT)
    return pl.pallas_call(
        _fmt_next_body,
        out_shape=jax.ShapeDtypeStruct((N_VT, N_PT, 8, 128), jnp.float32),
        grid=(PT_H // FPT,),
        in_specs=[
            pl.BlockSpec((FPT * 1024, 128), lambda i: (i, 0)),
            pl.BlockSpec(memory_space=pl.ANY),
        ],
        out_specs=pl.BlockSpec(
            (N_VT, FPT, 8, 128), lambda i: (0, off + i, 0, 0)
        ),
        input_output_aliases={1: 0},
    )(x3, o_prev)


@jax.jit
def _fmt_first_call(x3):
    return pl.pallas_call(
        _fmt_body,
        out_shape=jax.ShapeDtypeStruct((N_VT, N_PT, 8, 128), jnp.float32),
        grid=(PT_H // FPT,),
        in_specs=[pl.BlockSpec((FPT * 1024, 128), lambda i: (i, 0))],
        out_specs=pl.BlockSpec(
            (N_VT, FPT, 8, 128), lambda i: (0, i, 0, 0)
        ),
    )(x3)


def _loss_body(p_ref, o_ref):
    o_ref[...] = (jnp.sum(p_ref[...]) / N_TOK).reshape(1, 1)


@jax.jit
def _loss_call(partials):
    return pl.pallas_call(
        _loss_body,
        out_shape=jax.ShapeDtypeStruct((1, 1), jnp.float32),
    )(partials)


def kernel(idx, targets, token_emb):
    idx_f = idx.reshape(-1).astype(jnp.int32)
    t_f = targets.reshape(-1).astype(jnp.int32)
    lse, tpad = _lse_call(token_emb)
    lse_flat = lse.reshape(LSE_PAD)
    table16 = token_emb.reshape(VOCAB * VOCAB // 16, 16)

    lins = []
    parts = []
    for h in range(H):
        sl = slice(h * N_POS, (h + 1) * N_POS)
        lin_h, p_h = _sc_call(
            tpad, table16, idx_f[sl].reshape(NW, LW), t_f[sl], lse_flat
        )
        lins.append(lin_h)
        parts.append(p_h)

    out4 = _fmt_first_call(lins[0].reshape(N_POS * 8, 128))
    for h in range(1, H):
        out4 = _fmt_next_call(lins[h].reshape(N_POS * 8, 128), out4, h)

    logits2 = out4.transpose(1, 3, 0, 2).reshape(N_TOK, VOCAB)
    loss = _loss_call(jnp.concatenate(parts, axis=0))[0, 0]
    return logits2, loss
